# baseline (speedup 1.0000x reference)
import functools

import jax
import jax.numpy as jnp
from jax import lax
from jax.experimental import pallas as pl
from jax.experimental.pallas import tpu as pltpu
from jax.experimental.pallas import tpu_sc as plsc


_ROW_BLOCK = 256


def _round_to_bf16(x):
    u = lax.bitcast_convert_type(x, jnp.uint32)
    r = (u + jnp.uint32(0x7FFF) + ((u >> 16) & jnp.uint32(1))) & jnp.uint32(0xFFFF0000)
    return lax.bitcast_convert_type(r, jnp.float32)


def _argmin_body(z_ref, zr_ref, cb_ref, cbr_ref, idx_ref):
    zb = z_ref[...]
    cb = cb_ref[...]
    cross = lax.dot_general(
        zr_ref[...], cbr_ref[...],
        (((1,), (1,)), ((), ())),
        preferred_element_type=jnp.float32)
    c_sq = jnp.sum(cb * cb, axis=1)
    z_sq = jnp.sum(zb * zb, axis=1, keepdims=True)
    dist = z_sq - 2.0 * cross + c_sq[None, :]
    mn = jnp.min(dist, axis=1, keepdims=True)
    ids = lax.broadcasted_iota(jnp.int32, dist.shape, 1)
    idx = jnp.min(jnp.where(dist == mn, ids, jnp.int32(2**30)), axis=1)
    idx_ref[...] = idx.astype(jnp.int32)


def _argmin_indices(z2, codebook):
    rows, c = z2.shape
    n = codebook.shape[0]
    grid = rows // _ROW_BLOCK
    return pl.pallas_call(
        _argmin_body,
        grid=(grid,),
        in_specs=[
            pl.BlockSpec((_ROW_BLOCK, c), lambda i: (i, 0)),
            pl.BlockSpec((_ROW_BLOCK, c), lambda i: (i, 0)),
            pl.BlockSpec((n, c), lambda i: (0, 0)),
            pl.BlockSpec((n, c), lambda i: (0, 0)),
        ],
        out_specs=pl.BlockSpec((_ROW_BLOCK,), lambda i: (i,)),
        out_shape=jax.ShapeDtypeStruct((rows,), jnp.int32),
    )(z2, _round_to_bf16(z2).astype(jnp.bfloat16),
      codebook, _round_to_bf16(codebook).astype(jnp.bfloat16))


def _make_sc_gather(n, c, rows):
    info = plsc.get_sparse_core_info()
    nc, ns = info.num_cores, info.num_subcores
    nw = nc * ns
    b_per_w = rows // nw
    mesh = plsc.VectorSubcoreMesh(core_axis_name="c", subcore_axis_name="s")

    @functools.partial(
        pl.kernel,
        mesh=mesh,
        out_type=jax.ShapeDtypeStruct((rows, c), jnp.float32),
        scratch_types=[
            pltpu.VMEM((b_per_w,), jnp.int32),
            pltpu.VMEM((b_per_w + 16,), jnp.int32),
            pltpu.VMEM((b_per_w, c), jnp.float32),
            pltpu.SemaphoreType.DMA,
        ],
    )
    def gather_k(cb_hbm, idx_hbm, mask_hbm, out_hbm, idx_v, mask_v, rows_v, sem):
        wid = lax.axis_index("s") * nc + lax.axis_index("c")
        base = wid * b_per_w
        pltpu.sync_copy(idx_hbm.at[pl.ds(base, b_per_w)], idx_v)
        pltpu.sync_copy(mask_hbm.at[pl.ds(base, b_per_w)],
                        mask_v.at[pl.ds(0, b_per_w)])
        pltpu.async_copy(cb_hbm.at[idx_v], rows_v, sem).wait()

        zeros16 = jnp.zeros((16,), jnp.float32)

        def row_body(r, carry):
            m = mask_v[pl.ds(r, 16)][0]

            @pl.when(m == 0)
            def _():
                for ch in range(c // 16):
                    rows_v[r, pl.ds(ch * 16, 16)] = zeros16
            return carry

        lax.fori_loop(0, b_per_w, row_body, 0)
        pltpu.sync_copy(rows_v, out_hbm.at[pl.ds(base, b_per_w)])

    return gather_k


def kernel(z, mask, codebook):
    b, l, c = z.shape
    n = codebook.shape[0]
    rows = b * l
    z2 = z.reshape(rows, c)
    idx_flat = _argmin_indices(z2, codebook)
    mask_i = mask.reshape(rows).astype(jnp.int32)
    quant = _make_sc_gather(n, c, rows)(codebook, idx_flat, mask_i)
    return quant.reshape(b, l, c), idx_flat.reshape(b, l), 0.0
